# Initial kernel scaffold; baseline (speedup 1.0000x reference)
import jax
import jax.numpy as jnp
from jax.experimental import pallas as pl


def kernel(emb, edge_index, top_k, W1, b1, W2, b2):
    raise NotImplementedError("write your pallas kernel here")


# speedup vs baseline: 2.4630x; 2.4630x over previous
import functools

import jax
import jax.numpy as jnp
from jax import lax
from jax.experimental import pallas as pl
from jax.experimental.pallas import tpu as pltpu
from jax.experimental.pallas import tpu_sc as plsc

E = 320000
D = 256
H = 64
N_NODES = 10000


BLK = 4000
NB = E // BLK


def _mlp_body(emb_ref, w1_ref, b1_ref, w2_ref, b2_ref, out_ref):
    h = jnp.dot(emb_ref[...], w1_ref[...], preferred_element_type=jnp.float32)
    h = jnp.maximum(h + b1_ref[...], 0.0)
    lg = lax.dot_general(w2_ref[...], h, (((0,), (1,)), ((), ())),
                         preferred_element_type=jnp.float32)
    lg = lg + b2_ref[...]
    out_ref[...] = (1.0 / (1.0 + jnp.exp(-lg))).reshape(1, 1, BLK)


def _edge_scores(emb, W1, b1, W2, b2):
    out = pl.pallas_call(
        _mlp_body,
        grid=(NB,),
        in_specs=[
            pl.BlockSpec((BLK, D), lambda i: (i, 0)),
            pl.BlockSpec((D, H), lambda i: (0, 0)),
            pl.BlockSpec((1, H), lambda i: (0, 0)),
            pl.BlockSpec((H, 1), lambda i: (0, 0)),
            pl.BlockSpec((1, 1), lambda i: (0, 0)),
        ],
        out_specs=pl.BlockSpec((1, 1, BLK), lambda i: (i, 0, 0)),
        out_shape=jax.ShapeDtypeStruct((NB, 1, BLK), jnp.float32),
    )(emb, W1, b1.reshape(1, H), W2, b2.reshape(1, 1))
    return out.reshape(E)


NT = 16
CH = E // NT
NV = CH // 16
NPAD = 10240
NCH = NPAD // NT

_mesh = plsc.VectorSubcoreMesh(core_axis_name="c", subcore_axis_name="s",
                               num_cores=1)


@functools.partial(
    pl.kernel,
    out_type=(jax.ShapeDtypeStruct((E,), jnp.float32),
              jax.ShapeDtypeStruct((NPAD,), jnp.int32)),
    mesh=_mesh,
    compiler_params=pltpu.CompilerParams(needs_layout_passes=False),
    scratch_types=[
        pltpu.VMEM((CH,), jnp.float32),
        pltpu.VMEM((CH,), jnp.int32),
        pltpu.VMEM((CH,), jnp.int32),
        pltpu.VMEM((4096,), jnp.int32),
        pltpu.VMEM((256,), jnp.int32),
        pltpu.VMEM((16, 256), jnp.int32),
        pltpu.VMEM((NPAD,), jnp.int32),
        pltpu.VMEM((16, NCH), jnp.int32),
        pltpu.VMEM((NCH,), jnp.int32),
        pltpu.VMEM((16,), jnp.int32),
        pltpu.VMEM_SHARED((4, 16, 256), jnp.int32),
        pltpu.VMEM_SHARED((16, NPAD), jnp.int32),
    ],
)
def _select(mask_hbm, src_hbm, dst_hbm, rank_hbm, out_hbm, nodes_hbm,
            keys, srcv, dstv, hist16, merged, gbuf, node_loc, ngather, nout,
            rbuf, sh_hist, sh_nodes):
    sid = lax.axis_index("s")
    base = sid * CH

    pltpu.sync_copy(mask_hbm.at[pl.ds(base, CH)], keys)
    pltpu.sync_copy(src_hbm.at[pl.ds(base, CH)], srcv)
    pltpu.sync_copy(dst_hbm.at[pl.ds(base, CH)], dstv)
    pltpu.sync_copy(rank_hbm, rbuf)
    r_v = rbuf[...]

    lane_base = lax.iota(jnp.int32, 16) * 256
    ones = jnp.ones((16,), jnp.int32)
    zz = jnp.zeros((16,), jnp.int32)

    def _zero_nodes(i, _):
        node_loc[pl.ds(i * 16, 16)] = zz
        return 0
    lax.fori_loop(0, NPAD // 16, _zero_nodes, 0)

    prefix = jnp.int32(0)
    for p in range(4):
        shift = 24 - 8 * p
        shift_v = jnp.full((16,), shift, jnp.int32)
        eight_v = jnp.full((16,), 8, jnp.int32)
        prefix_v = jnp.full((16,), prefix, jnp.int32)

        def _zero_hist(i, _):
            hist16[pl.ds(i * 16, 16)] = zz
            return 0
        lax.fori_loop(0, 256, _zero_hist, 0)

        def _histo(i, _, _pv=prefix_v, _sv=shift_v, _p=p):
            v = keys[pl.ds(i * 16, 16)]
            bits = plsc.bitcast(v, jnp.int32)
            t = lax.shift_right_logical(bits, _sv)
            byte = jnp.bitwise_and(t, 255)
            idx = lane_base + byte
            if _p == 0:
                plsc.addupdate_scatter(hist16, [idx], ones)
            else:
                pred = lax.shift_right_logical(t, eight_v) == _pv
                plsc.addupdate_scatter(hist16, [idx], ones, mask=pred)
            return 0
        lax.fori_loop(0, NV, _histo, 0)

        def _lane_merge(j, _):
            acc = hist16[pl.ds(j * 16, 16)]
            for l in range(1, 16):
                acc = acc + hist16[pl.ds(l * 256 + j * 16, 16)]
            merged[pl.ds(j * 16, 16)] = acc
            return 0
        lax.fori_loop(0, 16, _lane_merge, 0)

        pltpu.sync_copy(merged, sh_hist.at[p, sid])
        plsc.subcore_barrier()
        pltpu.sync_copy(sh_hist.at[p], gbuf)

        def _tile_merge(j, _):
            acc = gbuf[0, pl.ds(j * 16, 16)]
            for t in range(1, 16):
                acc = acc + gbuf[t, pl.ds(j * 16, 16)]
            merged[pl.ds(j * 16, 16)] = acc
            return 0
        lax.fori_loop(0, 16, _tile_merge, 0)

        carry = jnp.zeros((16,), jnp.int32)
        ntrue = jnp.zeros((16,), jnp.int32)
        gt_acc = jnp.zeros((16,), jnp.int32)
        for j in reversed(range(16)):
            v = merged[pl.ds(j * 16, 16)]
            cum = plsc.cumsum(v)
            tot = cum[15]
            suffix = (tot - cum) + v + carry
            carry = carry + tot
            p = suffix > r_v
            ntrue = ntrue + plsc.all_reduce_population_count(p)
            gt_acc = gt_acc + jnp.where(p, zz, v)
        bn = ntrue[0] - 1
        r_v = r_v - jnp.sum(gt_acc)
        prefix = prefix * 256 + bn

    thr_v = plsc.bitcast(jnp.full((16,), prefix, jnp.int32), jnp.float32)
    zf = jnp.zeros((16,), jnp.float32)

    def _final(i, _):
        v = keys[pl.ds(i * 16, 16)]
        m = v > thr_v
        keys[pl.ds(i * 16, 16)] = jnp.where(m, v, zf)
        si = srcv[pl.ds(i * 16, 16)]
        plsc.store_scatter(node_loc, [si], ones, mask=m)
        di = dstv[pl.ds(i * 16, 16)]
        plsc.store_scatter(node_loc, [di], ones, mask=m)
        return 0
    lax.fori_loop(0, NV, _final, 0)

    pltpu.sync_copy(keys, out_hbm.at[pl.ds(base, CH)])
    pltpu.sync_copy(node_loc, sh_nodes.at[sid])
    plsc.subcore_barrier()

    nb = sid * NCH
    for t in range(16):
        pltpu.sync_copy(sh_nodes.at[t, pl.ds(nb, NCH)], ngather.at[t])

    def _node_merge(j, _):
        acc = ngather[0, pl.ds(j * 16, 16)]
        for t in range(1, 16):
            acc = jnp.bitwise_or(acc, ngather[t, pl.ds(j * 16, 16)])
        nout[pl.ds(j * 16, 16)] = acc
        return 0
    lax.fori_loop(0, NCH // 16, _node_merge, 0)
    pltpu.sync_copy(nout, nodes_hbm.at[pl.ds(nb, NCH)])


def kernel(emb, edge_index, top_k, W1, b1, W2, b2):
    edge_mask = _edge_scores(emb, W1, b1, W2, b2)
    rank = jnp.minimum(jnp.asarray(top_k, jnp.int32), E - 1)
    rank_arr = jnp.full((16,), rank, dtype=jnp.int32)
    masked, nodes = _select(edge_mask, edge_index[0], edge_index[1], rank_arr)
    return masked, nodes[:N_NODES].astype(bool)


# speedup vs baseline: 2.8553x; 1.1593x over previous
import functools

import jax
import jax.numpy as jnp
from jax import lax
from jax.experimental import pallas as pl
from jax.experimental.pallas import tpu as pltpu
from jax.experimental.pallas import tpu_sc as plsc

E = 320000
D = 256
H = 64
N_NODES = 10000


BLK = 8000
NB = E // BLK


def _mlp_body(emb_ref, w1_ref, b1_ref, w2_ref, b2_ref, out_ref):
    h = jnp.dot(emb_ref[...], w1_ref[...], preferred_element_type=jnp.float32)
    h = jnp.maximum(h + b1_ref[...], 0.0)
    lg = lax.dot_general(w2_ref[...], h, (((0,), (1,)), ((), ())),
                         preferred_element_type=jnp.float32)
    lg = lg + b2_ref[...]
    out_ref[...] = (1.0 / (1.0 + jnp.exp(-lg))).reshape(1, 1, BLK)


def _edge_scores(emb, W1, b1, W2, b2):
    out = pl.pallas_call(
        _mlp_body,
        grid=(NB,),
        in_specs=[
            pl.BlockSpec((BLK, D), lambda i: (i, 0)),
            pl.BlockSpec((D, H), lambda i: (0, 0)),
            pl.BlockSpec((1, H), lambda i: (0, 0)),
            pl.BlockSpec((H, 1), lambda i: (0, 0)),
            pl.BlockSpec((1, 1), lambda i: (0, 0)),
        ],
        out_specs=pl.BlockSpec((1, 1, BLK), lambda i: (i, 0, 0)),
        out_shape=jax.ShapeDtypeStruct((NB, 1, BLK), jnp.float32),
    )(emb, W1, b1.reshape(1, H), W2, b2.reshape(1, 1))
    return out.reshape(E)


NT = 16
CH = E // NT
NV = CH // 16
NPAD = 10240
NCH = NPAD // NT

_mesh = plsc.VectorSubcoreMesh(core_axis_name="c", subcore_axis_name="s",
                               num_cores=1)


@functools.partial(
    pl.kernel,
    out_type=(jax.ShapeDtypeStruct((E,), jnp.float32),
              jax.ShapeDtypeStruct((NPAD,), jnp.int32)),
    mesh=_mesh,
    compiler_params=pltpu.CompilerParams(needs_layout_passes=False),
    scratch_types=[
        pltpu.VMEM((CH,), jnp.float32),
        pltpu.VMEM((CH,), jnp.int32),
        pltpu.VMEM((CH,), jnp.int32),
        pltpu.VMEM((4096,), jnp.int32),
        pltpu.VMEM((256,), jnp.int32),
        pltpu.VMEM((16, 256), jnp.int32),
        pltpu.VMEM((NPAD,), jnp.int32),
        pltpu.VMEM((16, NCH), jnp.int32),
        pltpu.VMEM((NCH,), jnp.int32),
        pltpu.VMEM((16,), jnp.int32),
        pltpu.VMEM_SHARED((4, 16, 256), jnp.int32),
        pltpu.VMEM_SHARED((16, NPAD), jnp.int32),
        pltpu.SemaphoreType.DMA,
        pltpu.SemaphoreType.DMA,
    ],
)
def _select(mask_hbm, src_hbm, dst_hbm, rank_hbm, out_hbm, nodes_hbm,
            keys, srcv, dstv, hist16, merged, gbuf, node_loc, ngather, nout,
            rbuf, sh_hist, sh_nodes, sem_s, sem_d):
    sid = lax.axis_index("s")
    base = sid * CH

    d_src = pltpu.async_copy(src_hbm.at[pl.ds(base, CH)], srcv, sem_s)
    d_dst = pltpu.async_copy(dst_hbm.at[pl.ds(base, CH)], dstv, sem_d)
    pltpu.sync_copy(mask_hbm.at[pl.ds(base, CH)], keys)
    pltpu.sync_copy(rank_hbm, rbuf)
    r_v = rbuf[...]

    lane_base = lax.iota(jnp.int32, 16) * 256
    ones = jnp.ones((16,), jnp.int32)
    zz = jnp.zeros((16,), jnp.int32)

    def _zero_nodes(i, _):
        node_loc[pl.ds(i * 16, 16)] = zz
        return 0
    lax.fori_loop(0, NPAD // 16, _zero_nodes, 0)

    prefix = jnp.int32(0)
    for p in range(4):
        shift = 24 - 8 * p
        shift_v = jnp.full((16,), shift, jnp.int32)
        eight_v = jnp.full((16,), 8, jnp.int32)
        prefix_v = jnp.full((16,), prefix, jnp.int32)

        def _zero_hist(i, _):
            for u in range(4):
                hist16[pl.ds(i * 64 + u * 16, 16)] = zz
            return 0
        lax.fori_loop(0, 64, _zero_hist, 0)

        def _histo(i, _, _pv=prefix_v, _sv=shift_v, _p=p):
            for u in range(5):
                v = keys[pl.ds(i * 80 + u * 16, 16)]
                bits = plsc.bitcast(v, jnp.int32)
                t = lax.shift_right_logical(bits, _sv)
                byte = jnp.bitwise_and(t, 255)
                idx = lane_base + byte
                if _p == 0:
                    plsc.addupdate_scatter(hist16, [idx], ones)
                else:
                    pred = lax.shift_right_logical(t, eight_v) == _pv
                    plsc.addupdate_scatter(hist16, [idx], ones, mask=pred)
            return 0
        lax.fori_loop(0, NV // 5, _histo, 0)

        def _lane_merge(j, _):
            acc = hist16[pl.ds(j * 16, 16)]
            for l in range(1, 16):
                acc = acc + hist16[pl.ds(l * 256 + j * 16, 16)]
            merged[pl.ds(j * 16, 16)] = acc
            return 0
        lax.fori_loop(0, 16, _lane_merge, 0)

        pltpu.sync_copy(merged, sh_hist.at[p, sid])
        plsc.subcore_barrier()
        pltpu.sync_copy(sh_hist.at[p], gbuf)

        def _tile_merge(j, _):
            acc = gbuf[0, pl.ds(j * 16, 16)]
            for t in range(1, 16):
                acc = acc + gbuf[t, pl.ds(j * 16, 16)]
            merged[pl.ds(j * 16, 16)] = acc
            return 0
        lax.fori_loop(0, 16, _tile_merge, 0)

        carry = jnp.zeros((16,), jnp.int32)
        ntrue = jnp.zeros((16,), jnp.int32)
        gt_acc = jnp.zeros((16,), jnp.int32)
        for j in reversed(range(16)):
            v = merged[pl.ds(j * 16, 16)]
            cum = plsc.cumsum(v)
            tot = cum[15]
            suffix = (tot - cum) + v + carry
            carry = carry + tot
            p = suffix > r_v
            ntrue = ntrue + plsc.all_reduce_population_count(p)
            gt_acc = gt_acc + jnp.where(p, zz, v)
        bn = ntrue[0] - 1
        r_v = r_v - jnp.sum(gt_acc)
        prefix = prefix * 256 + bn

    thr_v = plsc.bitcast(jnp.full((16,), prefix, jnp.int32), jnp.float32)
    zf = jnp.zeros((16,), jnp.float32)

    d_src.wait()
    d_dst.wait()

    def _final(i, _):
        for u in range(5):
            o = i * 80 + u * 16
            v = keys[pl.ds(o, 16)]
            m = v > thr_v
            keys[pl.ds(o, 16)] = jnp.where(m, v, zf)
            si = srcv[pl.ds(o, 16)]
            plsc.store_scatter(node_loc, [si], ones, mask=m)
            di = dstv[pl.ds(o, 16)]
            plsc.store_scatter(node_loc, [di], ones, mask=m)
        return 0
    lax.fori_loop(0, NV // 5, _final, 0)

    pltpu.sync_copy(keys, out_hbm.at[pl.ds(base, CH)])
    pltpu.sync_copy(node_loc, sh_nodes.at[sid])
    plsc.subcore_barrier()

    nb = sid * NCH
    for t in range(16):
        pltpu.sync_copy(sh_nodes.at[t, pl.ds(nb, NCH)], ngather.at[t])

    def _node_merge(j, _):
        acc = ngather[0, pl.ds(j * 16, 16)]
        for t in range(1, 16):
            acc = jnp.bitwise_or(acc, ngather[t, pl.ds(j * 16, 16)])
        nout[pl.ds(j * 16, 16)] = acc
        return 0
    lax.fori_loop(0, NCH // 16, _node_merge, 0)
    pltpu.sync_copy(nout, nodes_hbm.at[pl.ds(nb, NCH)])


def kernel(emb, edge_index, top_k, W1, b1, W2, b2):
    edge_mask = _edge_scores(emb, W1, b1, W2, b2)
    rank = jnp.minimum(jnp.asarray(top_k, jnp.int32), E - 1)
    rank_arr = jnp.full((16,), rank, dtype=jnp.int32)
    masked, nodes = _select(edge_mask, edge_index[0], edge_index[1], rank_arr)
    return masked, nodes[:N_NODES].astype(bool)
